# baseline (speedup 1.0000x reference)
import functools

import jax
import jax.numpy as jnp
from jax import lax
from jax.experimental import pallas as pl
from jax.experimental.pallas import tpu as pltpu
from jax.experimental.pallas import tpu_sc as plsc

N_ROWS = 8192
N_COLS = 2048
NUM_WORKERS = 32
ROWS_PER_WORKER = N_ROWS // NUM_WORKERS
CHUNK_ROWS = 8
NUM_CHUNKS = ROWS_PER_WORKER // CHUNK_ROWS
NBUF = 6


def _sc_multiplex(x0, x1, x2, x3, sel_vec):
    mesh = plsc.VectorSubcoreMesh(core_axis_name="c", subcore_axis_name="s")

    @functools.partial(
        pl.kernel,
        mesh=mesh,
        out_type=jax.ShapeDtypeStruct((N_ROWS, N_COLS), jnp.float32),
        scratch_types=[
            pltpu.VMEM((16,), jnp.int32),
        ]
        + [pltpu.VMEM((CHUNK_ROWS, N_COLS), jnp.float32) for _ in range(NBUF)]
        + [pltpu.SemaphoreType.DMA for _ in range(2 * NBUF)],
    )
    def body(x0_h, x1_h, x2_h, x3_h, sel_h, out_h, sel_v, *bufs_and_sems):
        bufs = bufs_and_sems[:NBUF]
        rsem = bufs_and_sems[NBUF : 2 * NBUF]
        wsem = bufs_and_sems[2 * NBUF : 3 * NBUF]
        wid = lax.axis_index("s") * 2 + lax.axis_index("c")
        base = wid * ROWS_PER_WORKER
        pltpu.sync_copy(sel_h, sel_v)
        s = sel_v[...][0]

        def copy_from(src_h):
            D = 3

            def rd_wait(i):
                b = i % NBUF
                pltpu.make_async_copy(
                    src_h.at[pl.ds(base + i * CHUNK_ROWS, CHUNK_ROWS)],
                    bufs[b], rsem[b]).wait()

            def wr_wait(i):
                b = i % NBUF
                pltpu.make_async_copy(
                    bufs[b],
                    out_h.at[pl.ds(base + i * CHUNK_ROWS, CHUNK_ROWS)],
                    wsem[b]).wait()

            for i in range(NUM_CHUNKS + D):
                if i < NUM_CHUNKS:
                    b = i % NBUF
                    if i >= NBUF:
                        wr_wait(i - NBUF)
                    pltpu.async_copy(
                        src_h.at[pl.ds(base + i * CHUNK_ROWS, CHUNK_ROWS)],
                        bufs[b], rsem[b])
                if i >= D:
                    j = i - D
                    bj = j % NBUF
                    rd_wait(j)
                    pltpu.async_copy(
                        bufs[bj],
                        out_h.at[pl.ds(base + j * CHUNK_ROWS, CHUNK_ROWS)],
                        wsem[bj])
            for j in range(NUM_CHUNKS - NBUF, NUM_CHUNKS):
                wr_wait(j)

        for j, src in enumerate((x0_h, x1_h, x2_h, x3_h)):
            @pl.when(s == j)
            def _(src=src):
                copy_from(src)

    return body(x0, x1, x2, x3, sel_vec)


def kernel(x0, x1, x2, x3, sel):
    sel_vec = jnp.full((16,), sel, dtype=jnp.int32)
    return _sc_multiplex(x0, x1, x2, x3, sel_vec)


# speedup vs baseline: 1.5421x; 1.5421x over previous
import jax
import jax.numpy as jnp
from jax.experimental import pallas as pl
from jax.experimental.pallas import tpu as pltpu

N_ROWS = 8192
N_COLS = 2048
CHUNK_ROWS = 256
NUM_CHUNKS = N_ROWS // CHUNK_ROWS
NBUF = 8
D = 3


def _tc_multiplex(x0, x1, x2, x3, sel_arr):
    def body(sel_ref, x0_h, x1_h, x2_h, x3_h, out_h, *bufs_and_sems):
        bufs = bufs_and_sems[:NBUF]
        rsem = bufs_and_sems[NBUF : 2 * NBUF]
        wsem = bufs_and_sems[2 * NBUF : 3 * NBUF]
        s = sel_ref[0]

        def copy_from(src_h):
            def rd(i, wait):
                b = i % NBUF
                cp = pltpu.make_async_copy(
                    src_h.at[pl.ds(i * CHUNK_ROWS, CHUNK_ROWS)],
                    bufs[b], rsem[b])
                cp.wait() if wait else cp.start()

            def wr(i, wait):
                b = i % NBUF
                cp = pltpu.make_async_copy(
                    bufs[b],
                    out_h.at[pl.ds(i * CHUNK_ROWS, CHUNK_ROWS)],
                    wsem[b])
                cp.wait() if wait else cp.start()

            for i in range(NUM_CHUNKS + D):
                if i < NUM_CHUNKS:
                    if i >= NBUF:
                        wr(i - NBUF, True)
                    rd(i, False)
                if i >= D:
                    rd(i - D, True)
                    wr(i - D, False)
            for j in range(NUM_CHUNKS - NBUF, NUM_CHUNKS):
                wr(j, True)

        for j, src in enumerate((x0_h, x1_h, x2_h, x3_h)):
            @pl.when(s == j)
            def _(src=src):
                copy_from(src)

    return pl.pallas_call(
        body,
        in_specs=[
            pl.BlockSpec(memory_space=pltpu.SMEM),
            pl.BlockSpec(memory_space=pl.ANY),
            pl.BlockSpec(memory_space=pl.ANY),
            pl.BlockSpec(memory_space=pl.ANY),
            pl.BlockSpec(memory_space=pl.ANY),
        ],
        out_specs=pl.BlockSpec(memory_space=pl.ANY),
        out_shape=jax.ShapeDtypeStruct((N_ROWS, N_COLS), jnp.float32),
        scratch_shapes=(
            [pltpu.VMEM((CHUNK_ROWS, N_COLS), jnp.float32) for _ in range(NBUF)]
            + [pltpu.SemaphoreType.DMA for _ in range(2 * NBUF)]
        ),
    )(sel_arr, x0, x1, x2, x3)


def kernel(x0, x1, x2, x3, sel):
    sel_arr = jnp.asarray(sel, dtype=jnp.int32).reshape((1,))
    return _tc_multiplex(x0, x1, x2, x3, sel_arr)
